# baseline (speedup 1.0000x reference)
import jax
import jax.numpy as jnp
from jax.experimental import pallas as pl
from jax.experimental.pallas import tpu as pltpu

BSZ, N_HEADS, MAX_SEQ_LEN, HEAD_DIM = 8, 16, 4096, 64
SEQLEN = 16
BH = BSZ * N_HEADS
NCH = 8
CSZ = BH // NCH


def _body(pos_ref, c_ref, u_ref, o_ref, sems, usem):
    for k in range(NCH):
        pltpu.make_async_copy(
            c_ref.at[pl.ds(k * CSZ, CSZ)],
            o_ref.at[pl.ds(k * CSZ, CSZ)],
            sems.at[k],
        ).start()
    for k in range(NCH):
        pltpu.make_async_copy(
            c_ref.at[pl.ds(k * CSZ, CSZ)],
            o_ref.at[pl.ds(k * CSZ, CSZ)],
            sems.at[k],
        ).wait()
    p = pos_ref[0]
    up = pltpu.make_async_copy(u_ref, o_ref.at[:, pl.ds(p, SEQLEN), :], usem)
    up.start()
    up.wait()


def kernel(cache, update, pos):
    c3 = cache.reshape(BH, MAX_SEQ_LEN, HEAD_DIM)
    u3 = update.reshape(BH, SEQLEN, HEAD_DIM)
    out = pl.pallas_call(
        _body,
        grid_spec=pltpu.PrefetchScalarGridSpec(
            num_scalar_prefetch=1,
            grid=(1,),
            in_specs=[
                pl.BlockSpec(memory_space=pl.ANY),
                pl.BlockSpec(memory_space=pl.ANY),
            ],
            out_specs=pl.BlockSpec(memory_space=pl.ANY),
            scratch_shapes=[
                pltpu.SemaphoreType.DMA((NCH,)),
                pltpu.SemaphoreType.DMA,
            ],
        ),
        out_shape=jax.ShapeDtypeStruct((BH, MAX_SEQ_LEN, HEAD_DIM), cache.dtype),
    )(pos, c3, u3)
    return out.reshape(BSZ, N_HEADS, MAX_SEQ_LEN, HEAD_DIM)


# speedup vs baseline: 12.1205x; 12.1205x over previous
import jax
import jax.numpy as jnp
from jax.experimental import pallas as pl
from jax.experimental.pallas import tpu as pltpu

BSZ, N_HEADS, MAX_SEQ_LEN, HEAD_DIM = 8, 16, 4096, 64
SEQLEN = 16
BH = BSZ * N_HEADS
ROWS = MAX_SEQ_LEN * HEAD_DIM // 128
UROWS = SEQLEN * HEAD_DIM // 128
BLK_BH = 4


def _body(pos_ref, c_ref, u_ref, o_ref):
    o_ref[...] = c_ref[...]
    p = pos_ref[0]
    r0 = p // 2

    @pl.when(p % 2 == 0)
    def _even():
        o_ref[:, pl.ds(r0, UROWS), :] = u_ref[...]

    @pl.when(p % 2 == 1)
    def _odd():
        o_ref[:, pl.ds(r0, UROWS), 64:128] = u_ref[:, :, 0:64]
        o_ref[:, pl.ds(r0 + 1, UROWS), 0:64] = u_ref[:, :, 64:128]


def kernel(cache, update, pos):
    c3 = cache.reshape(BH, ROWS, 128)
    u3 = update.reshape(BH, UROWS, 128)
    out = pl.pallas_call(
        _body,
        grid_spec=pltpu.PrefetchScalarGridSpec(
            num_scalar_prefetch=1,
            grid=(BH // BLK_BH,),
            in_specs=[
                pl.BlockSpec((BLK_BH, ROWS, 128), lambda i, p: (i, 0, 0)),
                pl.BlockSpec((BLK_BH, UROWS, 128), lambda i, p: (i, 0, 0)),
            ],
            out_specs=pl.BlockSpec((BLK_BH, ROWS, 128), lambda i, p: (i, 0, 0)),
        ),
        out_shape=jax.ShapeDtypeStruct((BH, ROWS, 128), cache.dtype),
    )(pos, c3, u3)
    return out.reshape(BSZ, N_HEADS, MAX_SEQ_LEN, HEAD_DIM)
